# Initial kernel scaffold; baseline (speedup 1.0000x reference)
import functools

import jax
import jax.numpy as jnp
from jax import lax
from jax.experimental import pallas as pl
from jax.experimental.pallas import tpu as pltpu
from jax.experimental.pallas import tpu_sc as plsc

_NC = 2
_NS = 16
_NW = _NC * _NS

_CHUNK = 1024


def _make_gather(n_idx: int, vocab: int, d: int):
  per_w = n_idx // _NW
  n_chunks = per_w // _CHUNK
  assert per_w % _CHUNK == 0 and per_w % 8 == 0

  mesh = plsc.VectorSubcoreMesh(core_axis_name="c", subcore_axis_name="s")

  @functools.partial(
      pl.kernel,
      mesh=mesh,
      out_type=jax.ShapeDtypeStruct((n_idx, d), jnp.float32),
      scratch_types=[
          pltpu.VMEM((per_w,), jnp.int32),
          pltpu.VMEM((_CHUNK, d), jnp.float32),
          pltpu.SemaphoreType.DMA,
      ],
  )
  def gather_kernel(idx_hbm, table_hbm, out_hbm, idx_v, buf, gsem):
    wid = lax.axis_index("s") * _NC + lax.axis_index("c")
    base = wid * per_w
    pltpu.sync_copy(idx_hbm.at[pl.ds(base, per_w)], idx_v)
    for c in range(n_chunks):
      pltpu.async_copy(
          table_hbm.at[idx_v.at[pl.ds(c * _CHUNK, _CHUNK)]], buf, gsem
      ).wait()
      pltpu.sync_copy(buf, out_hbm.at[pl.ds(base + c * _CHUNK, _CHUNK)])

  return gather_kernel


def _proj_body(x_ref, w_ref, b_ref, s_ref, o_ref):
  p = jnp.dot(x_ref[...], w_ref[...], preferred_element_type=jnp.float32)
  o_ref[...] = jnp.cos(p + b_ref[...]) * jnp.sin(p) * s_ref[0, 0]


def _projection(packed, w4, b4, scale2):
  m = packed.shape[0]
  blk = 1024
  grid = (m // blk,)
  return pl.pallas_call(
      _proj_body,
      grid=grid,
      in_specs=[
          pl.BlockSpec((blk, 128), lambda i: (i, 0)),
          pl.BlockSpec((128, 128), lambda i: (0, 0)),
          pl.BlockSpec((1, 128), lambda i: (0, 0)),
          pl.BlockSpec((1, 1), lambda i: (0, 0)),
      ],
      out_specs=pl.BlockSpec((blk, 128), lambda i: (i, 0)),
      out_shape=jax.ShapeDtypeStruct((m, 128), jnp.float32),
  )(packed, w4, b4, scale2)


def kernel(x, table, W, bias, scale):
  b, l = x.shape
  vocab, d = table.shape
  n = b * l
  idx = x.reshape(n).astype(jnp.int32)

  gathered = _make_gather(n, vocab, d)(idx, table)

  pack = 128 // d
  packed = gathered.reshape(n // pack, pack * d)
  w4 = jnp.kron(jnp.eye(pack, dtype=W.dtype), W.T)
  b4 = jnp.tile(bias, (1, pack))
  scale2 = scale.reshape(1, 1)

  out = _projection(packed, w4, b4, scale2)
  return out.reshape(b, l, d)


# speedup vs baseline: 8.1635x; 8.1635x over previous
import functools

import jax
import jax.numpy as jnp
from jax import lax
from jax.experimental import pallas as pl
from jax.experimental.pallas import tpu as pltpu
from jax.experimental.pallas import tpu_sc as plsc

_NC = 2
_NS = 16
_NW = _NC * _NS

_CHUNK = 1024


def _make_gather(n_idx: int, vocab: int, d: int):
  per_w = n_idx // _NW
  n_chunks = per_w // _CHUNK
  assert per_w % _CHUNK == 0 and per_w % 8 == 0

  mesh = plsc.VectorSubcoreMesh(core_axis_name="c", subcore_axis_name="s")

  @functools.partial(
      pl.kernel,
      mesh=mesh,
      compiler_params=pltpu.CompilerParams(use_tc_tiling_on_sc=False),
      out_type=jax.ShapeDtypeStruct((n_idx, d), jnp.float32),
      scratch_types=[
          pltpu.VMEM((per_w,), jnp.int32),
          pltpu.VMEM((_CHUNK, d), jnp.float32),
          pltpu.SemaphoreType.DMA,
      ],
  )
  def gather_kernel(idx_hbm, table_hbm, out_hbm, idx_v, buf, gsem):
    wid = lax.axis_index("s") * _NC + lax.axis_index("c")
    base = wid * per_w
    pltpu.sync_copy(idx_hbm.at[pl.ds(base, per_w)], idx_v)
    for c in range(n_chunks):
      pltpu.async_copy(
          table_hbm.at[idx_v.at[pl.ds(c * _CHUNK, _CHUNK)]], buf, gsem
      ).wait()
      pltpu.sync_copy(buf, out_hbm.at[pl.ds(base + c * _CHUNK, _CHUNK)])

  return gather_kernel


def _proj_body(x_ref, w_ref, b_ref, s_ref, o_ref):
  p = jnp.dot(x_ref[...], w_ref[...], preferred_element_type=jnp.float32)
  o_ref[...] = jnp.cos(p + b_ref[...]) * jnp.sin(p) * s_ref[0, 0]


def _projection(packed, w4, b4, scale2):
  m = packed.shape[0]
  blk = 1024
  grid = (m // blk,)
  return pl.pallas_call(
      _proj_body,
      grid=grid,
      in_specs=[
          pl.BlockSpec((blk, 128), lambda i: (i, 0)),
          pl.BlockSpec((128, 128), lambda i: (0, 0)),
          pl.BlockSpec((1, 128), lambda i: (0, 0)),
          pl.BlockSpec((1, 1), lambda i: (0, 0)),
      ],
      out_specs=pl.BlockSpec((blk, 128), lambda i: (i, 0)),
      out_shape=jax.ShapeDtypeStruct((m, 128), jnp.float32),
  )(packed, w4, b4, scale2)


def kernel(x, table, W, bias, scale):
  b, l = x.shape
  vocab, d = table.shape
  n = b * l
  idx = x.reshape(n).astype(jnp.int32)

  gathered = _make_gather(n, vocab, d)(idx, table)

  pack = 128 // d
  packed = gathered.reshape(n // pack, pack * d)
  w4 = jnp.kron(jnp.eye(pack, dtype=W.dtype), W.T)
  b4 = jnp.tile(bias, (1, pack))
  scale2 = scale.reshape(1, 1)

  out = _projection(packed, w4, b4, scale2)
  return out.reshape(b, l, d)


# speedup vs baseline: 10.6106x; 1.2998x over previous
import functools

import jax
import jax.numpy as jnp
from jax import lax
from jax.experimental import pallas as pl
from jax.experimental.pallas import tpu as pltpu
from jax.experimental.pallas import tpu_sc as plsc

_NC = 2
_NS = 16
_NW = _NC * _NS

_CHUNK = 1024


def _make_gather(n_idx: int, d: int):
  per_w = n_idx // _NW
  n_chunks = per_w // _CHUNK
  assert per_w % _CHUNK == 0 and per_w % 8 == 0

  mesh = plsc.VectorSubcoreMesh(core_axis_name="c", subcore_axis_name="s")

  @functools.partial(
      pl.kernel,
      mesh=mesh,
      compiler_params=pltpu.CompilerParams(use_tc_tiling_on_sc=False),
      out_type=jax.ShapeDtypeStruct((n_idx, d), jnp.float32),
      scratch_types=[
          pltpu.VMEM((per_w,), jnp.int32),
          pltpu.VMEM((_CHUNK, d), jnp.float32),
          pltpu.SemaphoreType.DMA,
      ],
  )
  def gather_kernel(idx_hbm, table_hbm, out_hbm, idx_v, buf, gsem):
    wid = lax.axis_index("s") * _NC + lax.axis_index("c")
    base = wid * per_w
    pltpu.sync_copy(idx_hbm.at[pl.ds(base, per_w)], idx_v)
    for c in range(n_chunks):
      pltpu.async_copy(
          table_hbm.at[idx_v.at[pl.ds(c * _CHUNK, _CHUNK)]], buf, gsem
      ).wait()
      pltpu.sync_copy(buf, out_hbm.at[pl.ds(base + c * _CHUNK, _CHUNK)])

  return gather_kernel


def _make_proj_body(d: int, pack: int, b_quarter: int):
  def _proj_body(x_ref, w_ref, b_ref, s_ref, o_ref):
    w = w_ref[...]
    bias_col = b_ref[...]
    s = s_ref[0, 0]
    for j in range(pack):
      xj = x_ref[:, j * d:(j + 1) * d]
      p = lax.dot_general(
          w, xj, (((1,), (1,)), ((), ())),
          preferred_element_type=jnp.float32,
      )
      o_ref[0, :, j * b_quarter:(j + 1) * b_quarter] = (
          jnp.cos(p + bias_col) * jnp.sin(p) * s
      )
  return _proj_body


def _projection(packed, w, b_col, scale2, l_dim, b_dim):
  d = w.shape[0]
  pack = 128 // d
  b_quarter = b_dim // pack
  grid = (l_dim,)
  return pl.pallas_call(
      _make_proj_body(d, pack, b_quarter),
      grid=grid,
      in_specs=[
          pl.BlockSpec((b_quarter, 128), lambda l: (l, 0)),
          pl.BlockSpec((d, d), lambda l: (0, 0)),
          pl.BlockSpec((d, 1), lambda l: (0, 0)),
          pl.BlockSpec((1, 1), lambda l: (0, 0)),
      ],
      out_specs=pl.BlockSpec((1, d, b_dim), lambda l: (l, 0, 0)),
      out_shape=jax.ShapeDtypeStruct((l_dim, d, b_dim), jnp.float32),
  )(packed, w, b_col, scale2)


def kernel(x, table, W, bias, scale):
  b, l = x.shape
  d = table.shape[1]
  n = b * l
  pack = 128 // d
  idx_staged = x.T.reshape(l, pack, b // pack).transpose(0, 2, 1).reshape(n)

  gathered = _make_gather(n, d)(idx_staged, table)
  packed = gathered.reshape(n // pack, pack * d)

  b_col = bias.reshape(d, 1)
  scale2 = scale.reshape(1, 1)
  out_t = _projection(packed, W, b_col, scale2, l, b)
  return out_t.transpose(2, 0, 1)


# speedup vs baseline: 12.5645x; 1.1841x over previous
import functools

import jax
import jax.numpy as jnp
from jax import lax
from jax.experimental import pallas as pl
from jax.experimental.pallas import tpu as pltpu
from jax.experimental.pallas import tpu_sc as plsc

_NC = 2
_NS = 16
_NW = _NC * _NS

_CHUNK = 1024


def _make_gather(n_idx: int, d: int):
  per_w = n_idx // _NW
  n_chunks = per_w // _CHUNK
  assert per_w % _CHUNK == 0 and per_w % 8 == 0

  mesh = plsc.VectorSubcoreMesh(core_axis_name="c", subcore_axis_name="s")

  @functools.partial(
      pl.kernel,
      mesh=mesh,
      compiler_params=pltpu.CompilerParams(use_tc_tiling_on_sc=False),
      out_type=jax.ShapeDtypeStruct((n_idx, d), jnp.float32),
      scratch_types=[
          pltpu.VMEM((per_w,), jnp.int32),
          pltpu.VMEM((_CHUNK, d), jnp.float32),
          pltpu.SemaphoreType.DMA,
      ],
  )
  def gather_kernel(idx_hbm, table_hbm, out_hbm, idx_v, buf, gsem):
    wid = lax.axis_index("s") * _NC + lax.axis_index("c")
    base = wid * per_w
    pltpu.sync_copy(idx_hbm.at[pl.ds(base, per_w)], idx_v)
    for c in range(n_chunks):
      pltpu.async_copy(
          table_hbm.at[idx_v.at[pl.ds(c * _CHUNK, _CHUNK)]], buf, gsem
      ).wait()
      pltpu.sync_copy(buf, out_hbm.at[pl.ds(base + c * _CHUNK, _CHUNK)])

  return gather_kernel


def _make_proj_body(d: int, pack: int, b_quarter: int):
  def _proj_body(x_ref, w2_ref, b_ref, off_ref, hs_ref, o_ref):
    w2 = w2_ref[...]
    bias_col = b_ref[...]
    off_col = off_ref[...]
    hs = hs_ref[0, 0]
    for j in range(pack):
      xj = x_ref[:, j * d:(j + 1) * d]
      p2 = lax.dot_general(
          w2, xj, (((1,), (1,)), ((), ())),
          preferred_element_type=jnp.float32,
      )
      o_ref[0, :, j * b_quarter:(j + 1) * b_quarter] = (
          jnp.sin(p2 + bias_col) * hs - off_col
      )
  return _proj_body


def _projection(packed, w2, b_col, off_col, half_scale, l_dim, b_dim):
  d = w2.shape[0]
  pack = 128 // d
  b_quarter = b_dim // pack
  grid = (l_dim,)
  return pl.pallas_call(
      _make_proj_body(d, pack, b_quarter),
      grid=grid,
      in_specs=[
          pl.BlockSpec((b_quarter, 128), lambda l: (l, 0)),
          pl.BlockSpec((d, d), lambda l: (0, 0)),
          pl.BlockSpec((d, 1), lambda l: (0, 0)),
          pl.BlockSpec((d, 1), lambda l: (0, 0)),
          pl.BlockSpec((1, 1), lambda l: (0, 0)),
      ],
      out_specs=pl.BlockSpec((1, d, b_dim), lambda l: (l, 0, 0)),
      out_shape=jax.ShapeDtypeStruct((l_dim, d, b_dim), jnp.float32),
  )(packed, w2, b_col, off_col, half_scale)


def kernel(x, table, W, bias, scale):
  b, l = x.shape
  d = table.shape[1]
  n = b * l
  pack = 128 // d
  idx_staged = x.T.reshape(l, pack, b // pack).transpose(0, 2, 1).reshape(n)

  gathered = _make_gather(n, d)(idx_staged, table)
  packed = gathered.reshape(n // pack, pack * d)

  b_col = bias.reshape(d, 1)
  half_scale = (0.5 * scale).reshape(1, 1)
  off_col = jnp.sin(b_col) * half_scale
  out_t = _projection(packed, 2.0 * W, b_col, off_col, half_scale, l, b)
  return out_t.transpose(2, 0, 1)


# speedup vs baseline: 17.5495x; 1.3968x over previous
import functools

import jax
import jax.numpy as jnp
from jax import lax
from jax.experimental import pallas as pl
from jax.experimental.pallas import tpu as pltpu
from jax.experimental.pallas import tpu_sc as plsc

_NC = 2
_NS = 16
_NW = _NC * _NS

_CHUNK = 1024
_UBLK = 4096


def _make_stage_body(d: int, pack: int, ublk: int):
  def _stage_body(t_ref, w2_ref, b_ref, o_ref):
    w2 = w2_ref[...]
    bias_row = b_ref[...]
    for j in range(pack):
      tj = t_ref[:, j * ublk:(j + 1) * ublk]
      o_ref[:, j * d:(j + 1) * d] = lax.dot_general(
          tj, w2, (((0,), (1,)), ((), ())),
          preferred_element_type=jnp.float32,
      ) + bias_row
  return _stage_body


def _stage(table_t, w2, bias, v_dim, d):
  pack = 128 // d
  vq = pack * _UBLK
  n_blocks = -(-v_dim // vq)
  u_pad = n_blocks * _UBLK
  return pl.pallas_call(
      _make_stage_body(d, pack, _UBLK),
      grid=(n_blocks,),
      in_specs=[
          pl.BlockSpec((d, vq), lambda i: (0, i)),
          pl.BlockSpec((d, d), lambda i: (0, 0)),
          pl.BlockSpec((1, d), lambda i: (0, 0)),
      ],
      out_specs=pl.BlockSpec((_UBLK, 128), lambda i: (i, 0)),
      out_shape=jax.ShapeDtypeStruct((u_pad, 128), jnp.float32),
  )(table_t, w2, bias)


def _make_gather(n_idx: int, d: int):
  per_w = n_idx // _NW
  n_chunks = per_w // _CHUNK
  assert per_w % _CHUNK == 0 and per_w % 8 == 0

  mesh = plsc.VectorSubcoreMesh(core_axis_name="c", subcore_axis_name="s")

  @functools.partial(
      pl.kernel,
      mesh=mesh,
      compiler_params=pltpu.CompilerParams(use_tc_tiling_on_sc=False),
      out_type=jax.ShapeDtypeStruct((n_idx, d), jnp.float32),
      scratch_types=[
          pltpu.VMEM((per_w,), jnp.int32),
          pltpu.VMEM((_CHUNK, d), jnp.float32),
          pltpu.SemaphoreType.DMA,
      ],
  )
  def gather_kernel(idx_hbm, table_hbm, out_hbm, idx_v, buf, gsem):
    wid = lax.axis_index("s") * _NC + lax.axis_index("c")
    base = wid * per_w
    pltpu.sync_copy(idx_hbm.at[pl.ds(base, per_w)], idx_v)
    for c in range(n_chunks):
      pltpu.async_copy(
          table_hbm.at[idx_v.at[pl.ds(c * _CHUNK, _CHUNK)]], buf, gsem
      ).wait()
      pltpu.sync_copy(buf, out_hbm.at[pl.ds(base + c * _CHUNK, _CHUNK)])

  return gather_kernel


def _make_sine_body(d: int, pack: int, b_quarter: int):
  def _sine_body(x_ref, eye_ref, off_ref, hs_ref, o_ref):
    eye = eye_ref[...]
    off_col = off_ref[...]
    hs = hs_ref[0, 0]
    for j in range(pack):
      xj = x_ref[:, j * d:(j + 1) * d]
      p2 = lax.dot_general(
          eye, xj, (((1,), (1,)), ((), ())),
          preferred_element_type=jnp.float32,
      )
      o_ref[0, :, j * b_quarter:(j + 1) * b_quarter] = jnp.sin(p2) * hs - off_col
  return _sine_body


def _sine(packed, eye, off_col, half_scale, l_dim, b_dim, d):
  pack = 128 // d
  b_quarter = b_dim // pack
  return pl.pallas_call(
      _make_sine_body(d, pack, b_quarter),
      grid=(l_dim,),
      in_specs=[
          pl.BlockSpec((b_quarter, 128), lambda l: (l, 0)),
          pl.BlockSpec((d, d), lambda l: (0, 0)),
          pl.BlockSpec((d, 1), lambda l: (0, 0)),
          pl.BlockSpec((1, 1), lambda l: (0, 0)),
      ],
      out_specs=pl.BlockSpec((1, d, b_dim), lambda l: (l, 0, 0)),
      out_shape=jax.ShapeDtypeStruct((l_dim, d, b_dim), jnp.float32),
  )(packed, eye, off_col, half_scale)


def kernel(x, table, W, bias, scale):
  b, l = x.shape
  v_dim, d = table.shape
  n = b * l
  pack = 128 // d
  vq = pack * _UBLK

  stage = _stage(table.T, 2.0 * W, bias, v_dim, d)
  stage_rows = stage.reshape(stage.shape[0] * pack, d)

  idx_lb = x.T.reshape(l, pack, b // pack).transpose(0, 2, 1).reshape(n)
  rem = idx_lb % vq
  idx_staged = (idx_lb - rem) + pack * (rem % _UBLK) + rem // _UBLK

  gathered = _make_gather(n, d)(idx_staged, stage_rows)
  packed = gathered.reshape(n // pack, pack * d)

  half_scale = (0.5 * scale).reshape(1, 1)
  off_col = jnp.sin(bias.reshape(d, 1)) * half_scale
  eye = jnp.eye(d, dtype=jnp.float32)
  out_t = _sine(packed, eye, off_col, half_scale, l, b, d)
  return out_t.transpose(2, 0, 1)
